# baseline (speedup 1.0000x reference)
import functools

import jax
import jax.numpy as jnp
from jax import lax
from jax.experimental import pallas as pl
from jax.experimental.pallas import tpu as pltpu
from jax.experimental.pallas import tpu_sc as plsc

N = 10000
E = 320000
D = 128
LAYERS = 3
NC = 2
NS = 16
NW = NC * NS
CH = 128
EP = 327680
PCH = EP // CH
TPC = PCH // NW
PAIRS = TPC // 2
NP = 10240
SRP = NP // NS


@functools.lru_cache(maxsize=None)
def _sc_kernels():
    mesh = plsc.VectorSubcoreMesh(core_axis_name="c", subcore_axis_name="s",
                                  num_cores=NC, num_subcores=NS)

    @functools.partial(
        pl.kernel,
        out_type=jax.ShapeDtypeStruct((NC, NP, D), jnp.float32),
        mesh=mesh,
        scratch_types=[
            pltpu.VMEM_SHARED((NP, D), jnp.float32),
            pltpu.VMEM((2, 2, CH), jnp.int32),
            pltpu.VMEM((2, CH, D), jnp.float32),
            pltpu.SemaphoreType.DMA,
            pltpu.SemaphoreType.DMA,
            pltpu.SemaphoreType.DMA,
            pltpu.SemaphoreType.DMA,
        ],
    )
    def sc_agg(vals_hbm, src_hbm, dst_hbm, zero_hbm, out_hbm,
               acc, idx, rows, sI0, sI1, sG0, sG1):
        c = lax.axis_index("c")
        s = lax.axis_index("s")
        k = c * NS + s
        pltpu.sync_copy(zero_hbm.at[pl.ds(s * SRP, SRP)],
                        acc.at[pl.ds(s * SRP, SRP)])
        plsc.subcore_barrier()
        lo = k * TPC

        def issue_idx(j, slot, sem):
            pltpu.async_copy(src_hbm.at[pl.ds(j * CH, CH)], idx.at[slot, 0], sem)
            pltpu.async_copy(dst_hbm.at[pl.ds(j * CH, CH)], idx.at[slot, 1], sem)

        def wait_idx(slot, sem):
            pltpu.make_async_copy(src_hbm.at[pl.ds(0, CH)], idx.at[slot, 0], sem).wait()
            pltpu.make_async_copy(src_hbm.at[pl.ds(0, CH)], idx.at[slot, 1], sem).wait()

        def issue_gather(slot, sem):
            pltpu.async_copy(vals_hbm.at[idx.at[slot, 0]], rows.at[slot], sem)

        def wait_gather(slot, sem):
            pltpu.make_async_copy(vals_hbm.at[idx.at[slot, 0]], rows.at[slot], sem).wait()

        issue_idx(lo, 0, sI0)
        issue_idx(lo + 1, 1, sI1)
        wait_idx(0, sI0)
        issue_gather(0, sG0)

        def pair(g, carry):
            j0 = lo + 2 * g
            wait_gather(0, sG0)
            wait_idx(1, sI1)
            issue_gather(1, sG1)
            pltpu.sync_copy(rows.at[0], acc.at[idx.at[0, 1]], add=True)

            @pl.when(g < PAIRS - 1)
            def _():
                issue_idx(j0 + 2, 0, sI0)

            wait_gather(1, sG1)

            @pl.when(g < PAIRS - 1)
            def _():
                wait_idx(0, sI0)
                issue_gather(0, sG0)

            pltpu.sync_copy(rows.at[1], acc.at[idx.at[1, 1]], add=True)

            @pl.when(g < PAIRS - 1)
            def _():
                issue_idx(j0 + 3, 1, sI1)

            return carry

        lax.fori_loop(0, PAIRS, pair, 0)
        plsc.subcore_barrier()
        pltpu.sync_copy(acc.at[pl.ds(s * SRP, SRP)],
                        out_hbm.at[c, pl.ds(s * SRP, SRP)])

    @functools.partial(
        pl.kernel,
        out_type=jax.ShapeDtypeStruct((NC, NP, D), jnp.float32),
        mesh=mesh,
        scratch_types=[
            pltpu.VMEM_SHARED((NP, D), jnp.float32),
            pltpu.VMEM((2, CH), jnp.int32),
            pltpu.VMEM((CH, D), jnp.float32),
            pltpu.SemaphoreType.DMA,
            pltpu.SemaphoreType.DMA,
        ],
    )
    def sc_deg(dst_hbm, zero_hbm, out_hbm, acc, didx, ones, sI0, sI1):
        c = lax.axis_index("c")
        s = lax.axis_index("s")
        k = c * NS + s

        def fill(j, carry):
            ones[j, pl.ds(0, 16)] = jnp.ones((16,), jnp.float32)
            return carry

        lax.fori_loop(0, CH, fill, 0)
        pltpu.sync_copy(zero_hbm.at[pl.ds(s * SRP, SRP)],
                        acc.at[pl.ds(s * SRP, SRP)])
        plsc.subcore_barrier()
        lo = k * TPC

        def issue_idx(j, slot, sem):
            pltpu.async_copy(dst_hbm.at[pl.ds(j * CH, CH)], didx.at[slot], sem)

        def wait_idx(slot, sem):
            pltpu.make_async_copy(dst_hbm.at[pl.ds(0, CH)], didx.at[slot], sem).wait()

        issue_idx(lo, 0, sI0)
        issue_idx(lo + 1, 1, sI1)

        def pair(g, carry):
            j0 = lo + 2 * g
            wait_idx(0, sI0)
            pltpu.sync_copy(ones, acc.at[didx.at[0]], add=True)

            @pl.when(g < PAIRS - 1)
            def _():
                issue_idx(j0 + 2, 0, sI0)

            wait_idx(1, sI1)
            pltpu.sync_copy(ones, acc.at[didx.at[1]], add=True)

            @pl.when(g < PAIRS - 1)
            def _():
                issue_idx(j0 + 3, 1, sI1)

            return carry

        lax.fori_loop(0, PAIRS, pair, 0)
        plsc.subcore_barrier()
        pltpu.sync_copy(acc.at[pl.ds(s * SRP, SRP)],
                        out_hbm.at[c, pl.ds(s * SRP, SRP)])

    return sc_agg, sc_deg


def _dot(a, b):
    return lax.dot_general(a, b, (((1,), (0,)), ((), ())),
                           precision=lax.Precision.HIGHEST,
                           preferred_element_type=jnp.float32)


def _enc_body(x_ref, w_ref, b_ref, o_ref):
    o_ref[...] = jnp.maximum(_dot(x_ref[...], w_ref[...]) + b_ref[...], 0.0)


_enc = pl.pallas_call(
    _enc_body, out_shape=jax.ShapeDtypeStruct((N, D), jnp.float32))


def _layer_body(residual, h_ref, p_ref, degp_ref, w1h_ref, w1a_ref, b1_ref,
                w2_ref, b2_ref, o_ref):
    deg = degp_ref[0, :N, 0:1] + degp_ref[1, :N, 0:1]
    inv = 1.0 / jnp.maximum(deg, 1.0)
    agg = (p_ref[0, :N] + p_ref[1, :N]) * inv
    z = jnp.maximum(_dot(h_ref[...], w1h_ref[...]) +
                    _dot(agg, w1a_ref[...]) + b1_ref[...], 0.0)
    hn = jnp.maximum(_dot(z, w2_ref[...]) + b2_ref[...], 0.0)
    if residual:
        hn = hn + h_ref[...]
    o_ref[...] = hn


_layer_first = pl.pallas_call(
    functools.partial(_layer_body, False),
    out_shape=jax.ShapeDtypeStruct((N, D), jnp.float32))
_layer_res = pl.pallas_call(
    functools.partial(_layer_body, True),
    out_shape=jax.ShapeDtypeStruct((N, D), jnp.float32))


def _head_body(h_ref, w_ref, b_ref, o_ref):
    pooled = jnp.mean(h_ref[...], axis=0, keepdims=True)
    o_ref[...] = _dot(pooled, w_ref[...]) + b_ref[...]


_head = pl.pallas_call(
    _head_body, out_shape=jax.ShapeDtypeStruct((1, D), jnp.float32))


def kernel(x, edge_index, enc_W, enc_b, conv_W1, conv_b1, conv_W2, conv_b2,
           head_W, head_b):
    ei = edge_index.astype(jnp.int32)
    src1 = jnp.concatenate([ei[0], jnp.zeros((EP - E,), jnp.int32)])
    dst1 = jnp.concatenate([ei[1], jnp.full((EP - E,), N, jnp.int32)])
    zero_d = jnp.zeros((NP, D), jnp.float32)

    sc_agg, sc_deg = _sc_kernels()
    h = _enc(x, enc_W, enc_b.reshape(1, D))
    degp = sc_deg(dst1, zero_d)
    for i in range(LAYERS):
        part = sc_agg(h, src1, dst1, zero_d)
        layer = _layer_first if i == 0 else _layer_res
        h = layer(h, part, degp,
                  conv_W1[i, :D], conv_W1[i, D:], conv_b1[i].reshape(1, -1),
                  conv_W2[i], conv_b2[i].reshape(1, -1))
    out = _head(h, head_W, head_b.reshape(1, D))
    return out.reshape(D)


# speedup vs baseline: 1.0246x; 1.0246x over previous
import functools

import jax
import jax.numpy as jnp
from jax import lax
from jax.experimental import pallas as pl
from jax.experimental.pallas import tpu as pltpu
from jax.experimental.pallas import tpu_sc as plsc

N = 10000
E = 320000
D = 128
LAYERS = 3
NC = 2
NS = 16
NW = NC * NS
CH = 128
EP = 327680
PCH = EP // CH
TPC = PCH // NW
BLK = 8
HALF = TPC // BLK // 2
PCHP = PCH + 2 * BLK
NP = 10240
SRP = NP // NS


@functools.lru_cache(maxsize=None)
def _sc_kernels():
    mesh = plsc.VectorSubcoreMesh(core_axis_name="c", subcore_axis_name="s",
                                  num_cores=NC, num_subcores=NS)

    @functools.partial(
        pl.kernel,
        out_type=jax.ShapeDtypeStruct((NC, NP, D), jnp.float32),
        mesh=mesh,
        scratch_types=[
            pltpu.VMEM_SHARED((NP, D), jnp.float32),
            pltpu.VMEM((2, BLK, CH), jnp.int32),
            pltpu.VMEM((2, BLK, CH), jnp.int32),
            pltpu.VMEM((2, CH, D), jnp.float32),
            pltpu.SemaphoreType.DMA,
            pltpu.SemaphoreType.DMA,
            pltpu.SemaphoreType.DMA,
            pltpu.SemaphoreType.DMA,
        ],
    )
    def sc_agg(vals_hbm, src_hbm, dst_hbm, zero_hbm, out_hbm,
               acc, sidx, didx, rows, sI0, sI1, sG0, sG1):
        c = lax.axis_index("c")
        s = lax.axis_index("s")
        k = c * NS + s
        pltpu.sync_copy(zero_hbm.at[pl.ds(s * SRP, SRP)],
                        acc.at[pl.ds(s * SRP, SRP)])
        plsc.subcore_barrier()
        base = k * TPC
        sG = (sG0, sG1)
        sI = (sI0, sI1)

        def issue_idx(row, slot):
            pltpu.async_copy(src_hbm.at[pl.ds(row, BLK)], sidx.at[slot], sI[slot])
            pltpu.async_copy(dst_hbm.at[pl.ds(row, BLK)], didx.at[slot], sI[slot])

        def wait_idx(slot):
            pltpu.make_async_copy(src_hbm.at[pl.ds(0, BLK)], sidx.at[slot], sI[slot]).wait()
            pltpu.make_async_copy(src_hbm.at[pl.ds(0, BLK)], didx.at[slot], sI[slot]).wait()

        def process(slot):
            for sub in range(BLK // 2):
                ds_ = []
                for ci in range(2):
                    ds_.append(pltpu.async_copy(
                        vals_hbm.at[sidx.at[slot, sub * 2 + ci]],
                        rows.at[ci], sG[ci]))
                for ci in range(2):
                    ds_[ci].wait()
                    pltpu.sync_copy(rows.at[ci],
                                    acc.at[didx.at[slot, sub * 2 + ci]], add=True)

        issue_idx(base, 0)
        issue_idx(base + BLK, 1)

        def body(w, carry):
            r0 = base + 2 * w * BLK
            wait_idx(0)
            process(0)
            issue_idx(r0 + 2 * BLK, 0)
            wait_idx(1)
            process(1)
            issue_idx(r0 + 3 * BLK, 1)
            return carry

        lax.fori_loop(0, HALF, body, 0)
        wait_idx(0)
        wait_idx(1)
        plsc.subcore_barrier()
        pltpu.sync_copy(acc.at[pl.ds(s * SRP, SRP)],
                        out_hbm.at[c, pl.ds(s * SRP, SRP)])

    @functools.partial(
        pl.kernel,
        out_type=jax.ShapeDtypeStruct((NC, NP, D), jnp.float32),
        mesh=mesh,
        scratch_types=[
            pltpu.VMEM_SHARED((NP, D), jnp.float32),
            pltpu.VMEM((2, BLK, CH), jnp.int32),
            pltpu.VMEM((CH, D), jnp.float32),
            pltpu.SemaphoreType.DMA,
            pltpu.SemaphoreType.DMA,
        ],
    )
    def sc_deg(dst_hbm, zero_hbm, out_hbm, acc, didx, ones, sI0, sI1):
        c = lax.axis_index("c")
        s = lax.axis_index("s")
        k = c * NS + s

        def fill(j, carry):
            ones[j, pl.ds(0, 16)] = jnp.ones((16,), jnp.float32)
            return carry

        lax.fori_loop(0, CH, fill, 0)
        pltpu.sync_copy(zero_hbm.at[pl.ds(s * SRP, SRP)],
                        acc.at[pl.ds(s * SRP, SRP)])
        plsc.subcore_barrier()
        base = k * TPC
        sI = (sI0, sI1)

        def issue_idx(row, slot):
            pltpu.async_copy(dst_hbm.at[pl.ds(row, BLK)], didx.at[slot], sI[slot])

        def wait_idx(slot):
            pltpu.make_async_copy(dst_hbm.at[pl.ds(0, BLK)], didx.at[slot], sI[slot]).wait()

        def process(slot):
            for ci in range(BLK):
                pltpu.sync_copy(ones, acc.at[didx.at[slot, ci]], add=True)

        issue_idx(base, 0)
        issue_idx(base + BLK, 1)

        def body(w, carry):
            r0 = base + 2 * w * BLK
            wait_idx(0)
            process(0)
            issue_idx(r0 + 2 * BLK, 0)
            wait_idx(1)
            process(1)
            issue_idx(r0 + 3 * BLK, 1)
            return carry

        lax.fori_loop(0, HALF, body, 0)
        wait_idx(0)
        wait_idx(1)
        plsc.subcore_barrier()
        pltpu.sync_copy(acc.at[pl.ds(s * SRP, SRP)],
                        out_hbm.at[c, pl.ds(s * SRP, SRP)])

    return sc_agg, sc_deg


def _dot(a, b):
    return lax.dot_general(a, b, (((1,), (0,)), ((), ())),
                           precision=lax.Precision.HIGHEST,
                           preferred_element_type=jnp.float32)


def _enc_body(x_ref, w_ref, b_ref, o_ref):
    o_ref[...] = jnp.maximum(_dot(x_ref[...], w_ref[...]) + b_ref[...], 0.0)


_enc = pl.pallas_call(
    _enc_body, out_shape=jax.ShapeDtypeStruct((N, D), jnp.float32))


def _layer_body(residual, h_ref, p_ref, degp_ref, w1h_ref, w1a_ref, b1_ref,
                w2_ref, b2_ref, o_ref):
    deg = degp_ref[0, :N, 0:1] + degp_ref[1, :N, 0:1]
    inv = 1.0 / jnp.maximum(deg, 1.0)
    agg = (p_ref[0, :N] + p_ref[1, :N]) * inv
    z = jnp.maximum(_dot(h_ref[...], w1h_ref[...]) +
                    _dot(agg, w1a_ref[...]) + b1_ref[...], 0.0)
    hn = jnp.maximum(_dot(z, w2_ref[...]) + b2_ref[...], 0.0)
    if residual:
        hn = hn + h_ref[...]
    o_ref[...] = hn


_layer_first = pl.pallas_call(
    functools.partial(_layer_body, False),
    out_shape=jax.ShapeDtypeStruct((N, D), jnp.float32))
_layer_res = pl.pallas_call(
    functools.partial(_layer_body, True),
    out_shape=jax.ShapeDtypeStruct((N, D), jnp.float32))


def _head_body(h_ref, w_ref, b_ref, o_ref):
    pooled = jnp.mean(h_ref[...], axis=0, keepdims=True)
    o_ref[...] = _dot(pooled, w_ref[...]) + b_ref[...]


_head = pl.pallas_call(
    _head_body, out_shape=jax.ShapeDtypeStruct((1, D), jnp.float32))


def kernel(x, edge_index, enc_W, enc_b, conv_W1, conv_b1, conv_W2, conv_b2,
           head_W, head_b):
    ei = edge_index.astype(jnp.int32)
    epad = PCHP * CH
    src2 = jnp.concatenate(
        [ei[0], jnp.zeros((epad - E,), jnp.int32)]).reshape(PCHP, CH)
    dst2 = jnp.concatenate(
        [ei[1], jnp.full((EP - E,), N, jnp.int32),
         jnp.zeros((epad - EP,), jnp.int32)]).reshape(PCHP, CH)
    zero_d = jnp.zeros((NP, D), jnp.float32)

    sc_agg, sc_deg = _sc_kernels()
    h = _enc(x, enc_W, enc_b.reshape(1, D))
    degp = sc_deg(dst2, zero_d)
    for i in range(LAYERS):
        part = sc_agg(h, src2, dst2, zero_d)
        layer = _layer_first if i == 0 else _layer_res
        h = layer(h, part, degp,
                  conv_W1[i, :D], conv_W1[i, D:], conv_b1[i].reshape(1, -1),
                  conv_W2[i], conv_b2[i].reshape(1, -1))
    out = _head(h, head_W, head_b.reshape(1, D))
    return out.reshape(D)
